# baseline (speedup 1.0000x reference)
import functools

import jax
import jax.numpy as jnp
from jax.experimental import pallas as pl
from jax.experimental.pallas import tpu as pltpu


def _pick_tile(batch):
    for cand in (4096, 2048, 1024, 512, 256, 128, 64, 32, 16, 8):
        if cand < batch and batch % cand == 0:
            return cand
    return batch


def _multihot_lookup_kernel(x_ref, p_ref, t_ref, o_ref, *, vpf, num_fields):
    xv = jnp.clip(x_ref[...], 0, vpf - 1).astype(jnp.float32)
    xrep = jnp.dot(xv, p_ref[...], preferred_element_type=jnp.float32)
    lane = jax.lax.broadcasted_iota(jnp.int32, xrep.shape, 1)
    cand = (lane // num_fields).astype(jnp.float32)
    m = jnp.where(xrep == cand, jnp.float32(1.0), jnp.float32(0.0))
    o_ref[...] = jnp.dot(m, t_ref[...], preferred_element_type=jnp.float32)


def kernel(x, table_cat, field_offsets, field_num):
    batch, num_fields = x.shape
    v_total, emb_dim = table_cat.shape
    vpf = v_total // num_fields
    out_w = num_fields * emb_dim

    v = jnp.arange(v_total)
    fld = v % num_fields
    u = v // num_fields
    f_ids = jnp.arange(num_fields)

    p_mat = (fld[None, :] == f_ids[:, None]).astype(jnp.float32)

    src = field_offsets.astype(jnp.int32)[fld] + jnp.minimum(
        u.astype(jnp.int32), field_num.astype(jnp.int32)[fld])
    rows = table_cat[src]
    colmask = (fld[:, None] == f_ids[None, :]).astype(table_cat.dtype)
    t_big = (rows[:, None, :] * colmask[:, :, None]).reshape(v_total, out_w)

    bt = _pick_tile(batch)
    out_flat = pl.pallas_call(
        functools.partial(_multihot_lookup_kernel, vpf=vpf,
                          num_fields=num_fields),
        out_shape=jax.ShapeDtypeStruct((batch, out_w), table_cat.dtype),
        grid=(batch // bt,),
        in_specs=[
            pl.BlockSpec((bt, num_fields), lambda b: (b, 0)),
            pl.BlockSpec((num_fields, v_total), lambda b: (0, 0)),
            pl.BlockSpec((v_total, out_w), lambda b: (0, 0)),
        ],
        out_specs=pl.BlockSpec((bt, out_w), lambda b: (b, 0)),
        compiler_params=pltpu.CompilerParams(
            dimension_semantics=("parallel",)),
    )(x, p_mat, t_big)
    return out_flat.reshape(batch, num_fields, emb_dim)


# speedup vs baseline: 1.0304x; 1.0304x over previous
import functools

import jax
import jax.numpy as jnp
from jax.experimental import pallas as pl
from jax.experimental.pallas import tpu as pltpu


def _pick_tile(batch):
    for cand in (4096, 2048, 1024, 512, 256, 128, 64, 32, 16, 8):
        if cand < batch and batch % cand == 0:
            return cand
    return batch


def _multihot_lookup_kernel(x_ref, p_ref, t_ref, o_ref, *, vpf, num_fields):
    bt = x_ref.shape[0]
    v_total = t_ref.shape[0]
    rep_w = p_ref.shape[1]
    ngroups = v_total // rep_w
    per_group = rep_w // num_fields
    xv = jnp.clip(x_ref[...], 0, vpf - 1).astype(jnp.float32)
    xsmall = jnp.dot(xv, p_ref[...], preferred_element_type=jnp.float32)
    lane = jax.lax.broadcasted_iota(jnp.int32, (bt, rep_w), 1)
    cand0 = (lane // num_fields).astype(jnp.float32)
    m = jnp.concatenate(
        [jnp.where(xsmall == cand0 + jnp.float32(g * per_group),
                   jnp.float32(1.0), jnp.float32(0.0))
         for g in range(ngroups)], axis=1)
    o_ref[...] = jnp.dot(m, t_ref[...], preferred_element_type=jnp.float32)


def kernel(x, table_cat, field_offsets, field_num):
    batch, num_fields = x.shape
    v_total, emb_dim = table_cat.shape
    vpf = v_total // num_fields
    out_w = num_fields * emb_dim

    v = jnp.arange(v_total)
    fld = v % num_fields
    u = v // num_fields
    f_ids = jnp.arange(num_fields)

    rep_w = 128
    j = jnp.arange(rep_w)
    p_mat = ((j % num_fields)[None, :] == f_ids[:, None]).astype(jnp.float32)

    src = field_offsets.astype(jnp.int32)[fld] + jnp.minimum(
        u.astype(jnp.int32), field_num.astype(jnp.int32)[fld])
    rows = table_cat[src]
    colmask = (fld[:, None] == f_ids[None, :]).astype(table_cat.dtype)
    t_big = (rows[:, None, :] * colmask[:, :, None]).reshape(v_total, out_w)

    bt = _pick_tile(batch)
    out_flat = pl.pallas_call(
        functools.partial(_multihot_lookup_kernel, vpf=vpf,
                          num_fields=num_fields),
        out_shape=jax.ShapeDtypeStruct((batch, out_w), table_cat.dtype),
        grid=(batch // bt,),
        in_specs=[
            pl.BlockSpec((bt, num_fields), lambda b: (b, 0)),
            pl.BlockSpec((num_fields, rep_w), lambda b: (0, 0)),
            pl.BlockSpec((v_total, out_w), lambda b: (0, 0)),
        ],
        out_specs=pl.BlockSpec((bt, out_w), lambda b: (b, 0)),
        compiler_params=pltpu.CompilerParams(
            dimension_semantics=("parallel",)),
    )(x, p_mat, t_big)
    return out_flat.reshape(batch, num_fields, emb_dim)


# speedup vs baseline: 3.4213x; 3.3205x over previous
import functools

import jax
import jax.numpy as jnp
from jax.experimental import pallas as pl
from jax.experimental.pallas import tpu as pltpu


def _pick_tile(batch):
    for cand in (4096, 2048, 1024, 512, 256, 128):
        if cand < batch and batch % cand == 0:
            return cand
    return batch


def _multihot_lookup_t_kernel(xt_ref, tt_ref, ot_ref, *, vpf, num_fields):
    xv = jnp.clip(xt_ref[...], 0, vpf - 1).astype(jnp.float32)
    xrep = jnp.concatenate([xv] * vpf, axis=0)
    row = jax.lax.broadcasted_iota(jnp.int32, xrep.shape, 0)
    cand = (row // num_fields).astype(jnp.float32)
    m_t = jnp.where(xrep == cand, jnp.float32(1.0), jnp.float32(0.0))
    ot_ref[...] = jnp.dot(tt_ref[...], m_t,
                          preferred_element_type=jnp.float32)


def kernel(x, table_cat, field_offsets, field_num):
    batch, num_fields = x.shape
    v_total, emb_dim = table_cat.shape
    vpf = v_total // num_fields
    out_w = num_fields * emb_dim

    v = jnp.arange(v_total)
    fld = v % num_fields
    u = v // num_fields
    f_ids = jnp.arange(num_fields)

    src = field_offsets.astype(jnp.int32)[fld] + jnp.minimum(
        u.astype(jnp.int32), field_num.astype(jnp.int32)[fld])
    rows_t = table_cat[src].T
    rowmask = (f_ids[:, None] == fld[None, :]).astype(table_cat.dtype)
    t_big_t = (rowmask[:, None, :] * rows_t[None, :, :]).reshape(out_w, v_total)

    x_t = x.T
    bt = _pick_tile(batch)
    out_t = pl.pallas_call(
        functools.partial(_multihot_lookup_t_kernel, vpf=vpf,
                          num_fields=num_fields),
        out_shape=jax.ShapeDtypeStruct((out_w, batch), table_cat.dtype),
        grid=(batch // bt,),
        in_specs=[
            pl.BlockSpec((num_fields, bt), lambda b: (0, b)),
            pl.BlockSpec((out_w, v_total), lambda b: (0, 0)),
        ],
        out_specs=pl.BlockSpec((out_w, bt), lambda b: (0, b)),
        compiler_params=pltpu.CompilerParams(
            dimension_semantics=("parallel",)),
    )(x_t, t_big_t)
    return out_t.reshape(num_fields, emb_dim, batch).transpose(2, 0, 1)


# speedup vs baseline: 4.1383x; 1.2096x over previous
import functools

import jax
import jax.numpy as jnp
from jax.experimental import pallas as pl
from jax.experimental.pallas import tpu as pltpu


def _pick_tile(batch):
    for cand in (8192, 4096, 2048, 1024, 512, 256, 128):
        if cand < batch and batch % cand == 0:
            return cand
    return batch


def _multihot_lookup_t_kernel(xt_ref, tt_ref, ot_ref, *, vpf, num_fields):
    xv = jnp.clip(xt_ref[...], 0, vpf - 1).astype(jnp.float32)
    xrep = jnp.concatenate([xv] * vpf, axis=0)
    row = jax.lax.broadcasted_iota(jnp.int32, xrep.shape, 0)
    cand = (row // num_fields).astype(jnp.float32)
    m_t = jnp.where(xrep == cand, jnp.float32(1.0), jnp.float32(0.0))
    ot_ref[...] = jnp.dot(tt_ref[...], m_t,
                          preferred_element_type=jnp.float32)


def kernel(x, table_cat, field_offsets, field_num):
    batch, num_fields = x.shape
    v_total, emb_dim = table_cat.shape
    vpf = v_total // num_fields
    out_w = num_fields * emb_dim

    v = jnp.arange(v_total)
    fld = v % num_fields
    u = v // num_fields
    f_ids = jnp.arange(num_fields)

    src = field_offsets.astype(jnp.int32)[fld] + jnp.minimum(
        u.astype(jnp.int32), field_num.astype(jnp.int32)[fld])
    rows_t = table_cat[src].T
    rowmask = (f_ids[:, None] == fld[None, :]).astype(table_cat.dtype)
    t_big_t = (rowmask[:, None, :] * rows_t[None, :, :]).reshape(out_w, v_total)

    x_t = x.T
    bt = _pick_tile(batch)
    out_t = pl.pallas_call(
        functools.partial(_multihot_lookup_t_kernel, vpf=vpf,
                          num_fields=num_fields),
        out_shape=jax.ShapeDtypeStruct((out_w, batch), table_cat.dtype),
        grid=(batch // bt,),
        in_specs=[
            pl.BlockSpec((num_fields, bt), lambda b: (0, b)),
            pl.BlockSpec((out_w, v_total), lambda b: (0, 0)),
        ],
        out_specs=pl.BlockSpec((out_w, bt), lambda b: (0, b)),
        compiler_params=pltpu.CompilerParams(
            dimension_semantics=("parallel",)),
    )(x_t, t_big_t)
    return out_t.reshape(num_fields, emb_dim, batch).transpose(2, 0, 1)


# speedup vs baseline: 4.5806x; 1.1069x over previous
import functools

import jax
import jax.numpy as jnp
from jax.experimental import pallas as pl
from jax.experimental.pallas import tpu as pltpu


def _pick_tile(batch):
    for cand in (16384, 8192, 4096, 2048, 1024, 512, 256, 128):
        if cand < batch and batch % cand == 0:
            return cand
    return batch


def _multihot_lookup_t_kernel(xt_ref, tt_ref, ot_ref, *, vpf, num_fields):
    xv = jnp.clip(xt_ref[...], 0, vpf - 1).astype(jnp.float32)
    xrep = jnp.concatenate([xv] * vpf, axis=0)
    row = jax.lax.broadcasted_iota(jnp.int32, xrep.shape, 0)
    cand = (row // num_fields).astype(jnp.float32)
    m_t = jnp.where(xrep == cand, jnp.float32(1.0), jnp.float32(0.0))
    ot_ref[...] = jnp.dot(tt_ref[...], m_t,
                          preferred_element_type=jnp.float32)


def kernel(x, table_cat, field_offsets, field_num):
    batch, num_fields = x.shape
    v_total, emb_dim = table_cat.shape
    vpf = v_total // num_fields
    out_w = num_fields * emb_dim

    v = jnp.arange(v_total)
    fld = v % num_fields
    u = v // num_fields
    f_ids = jnp.arange(num_fields)

    src = field_offsets.astype(jnp.int32)[fld] + jnp.minimum(
        u.astype(jnp.int32), field_num.astype(jnp.int32)[fld])
    rows_t = table_cat[src].T
    rowmask = (f_ids[:, None] == fld[None, :]).astype(table_cat.dtype)
    t_big_t = (rowmask[:, None, :] * rows_t[None, :, :]).reshape(out_w, v_total)

    x_t = x.T
    bt = _pick_tile(batch)
    out_t = pl.pallas_call(
        functools.partial(_multihot_lookup_t_kernel, vpf=vpf,
                          num_fields=num_fields),
        out_shape=jax.ShapeDtypeStruct((out_w, batch), table_cat.dtype),
        grid=(batch // bt,),
        in_specs=[
            pl.BlockSpec((num_fields, bt), lambda b: (0, b)),
            pl.BlockSpec((out_w, v_total), lambda b: (0, 0)),
        ],
        out_specs=pl.BlockSpec((out_w, bt), lambda b: (0, b)),
        compiler_params=pltpu.CompilerParams(
            dimension_semantics=("parallel",)),
    )(x_t, t_big_t)
    return out_t.reshape(num_fields, emb_dim, batch).transpose(2, 0, 1)


# speedup vs baseline: 4.6032x; 1.0049x over previous
import functools

import jax
import jax.numpy as jnp
from jax.experimental import pallas as pl
from jax.experimental.pallas import tpu as pltpu


def _pick_tile(batch):
    for cand in (16384, 8192, 4096, 2048, 1024, 512, 256, 128):
        if cand < batch and batch % cand == 0:
            return cand
    return batch


def _multihot_lookup_t_kernel(xt_ref, tt_ref, ot_ref, *, vpf, num_fields):
    xv = jnp.clip(xt_ref[...], 0, vpf - 1).astype(jnp.float32)
    xrep = jnp.concatenate([xv] * vpf, axis=0)
    row = jax.lax.broadcasted_iota(jnp.int32, xrep.shape, 0)
    cand = (row // num_fields).astype(jnp.float32)
    m_t = (xrep == cand).astype(jnp.bfloat16)
    ot_ref[...] = jnp.dot(tt_ref[...], m_t,
                          preferred_element_type=jnp.float32)


def kernel(x, table_cat, field_offsets, field_num):
    batch, num_fields = x.shape
    v_total, emb_dim = table_cat.shape
    vpf = v_total // num_fields
    out_w = num_fields * emb_dim

    v = jnp.arange(v_total)
    fld = v % num_fields
    u = v // num_fields
    f_ids = jnp.arange(num_fields)

    src = field_offsets.astype(jnp.int32)[fld] + jnp.minimum(
        u.astype(jnp.int32), field_num.astype(jnp.int32)[fld])
    rows_t = table_cat[src].T
    rowmask = (f_ids[:, None] == fld[None, :]).astype(table_cat.dtype)
    t_big_t = (rowmask[:, None, :] * rows_t[None, :, :]).reshape(out_w, v_total)
    t_big_t = t_big_t.astype(jnp.bfloat16)

    x_t = x.T
    bt = _pick_tile(batch)
    out_t = pl.pallas_call(
        functools.partial(_multihot_lookup_t_kernel, vpf=vpf,
                          num_fields=num_fields),
        out_shape=jax.ShapeDtypeStruct((out_w, batch), table_cat.dtype),
        grid=(batch // bt,),
        in_specs=[
            pl.BlockSpec((num_fields, bt), lambda b: (0, b)),
            pl.BlockSpec((out_w, v_total), lambda b: (0, 0)),
        ],
        out_specs=pl.BlockSpec((out_w, bt), lambda b: (0, b)),
        compiler_params=pltpu.CompilerParams(
            dimension_semantics=("parallel",)),
    )(x_t, t_big_t)
    return out_t.reshape(num_fields, emb_dim, batch).transpose(2, 0, 1)


# speedup vs baseline: 4.6776x; 1.0162x over previous
import functools

import jax
import jax.numpy as jnp
from jax.experimental import pallas as pl
from jax.experimental.pallas import tpu as pltpu


def _pick_tile(batch):
    for cand in (32768, 16384, 8192, 4096, 2048, 1024, 512, 256, 128):
        if cand < batch and batch % cand == 0:
            return cand
    return batch


def _multihot_lookup_t_kernel(xt_ref, tt_ref, ot_ref, *, vpf, num_fields):
    xv = jnp.clip(xt_ref[...], 0, vpf - 1).astype(jnp.float32)
    xrep = jnp.concatenate([xv] * vpf, axis=0)
    row = jax.lax.broadcasted_iota(jnp.int32, xrep.shape, 0)
    cand = (row // num_fields).astype(jnp.float32)
    m_t = (xrep == cand).astype(jnp.bfloat16)
    ot_ref[...] = jnp.dot(tt_ref[...], m_t,
                          preferred_element_type=jnp.float32)


def kernel(x, table_cat, field_offsets, field_num):
    batch, num_fields = x.shape
    v_total, emb_dim = table_cat.shape
    vpf = v_total // num_fields
    out_w = num_fields * emb_dim

    v = jnp.arange(v_total)
    fld = v % num_fields
    u = v // num_fields
    f_ids = jnp.arange(num_fields)

    src = field_offsets.astype(jnp.int32)[fld] + jnp.minimum(
        u.astype(jnp.int32), field_num.astype(jnp.int32)[fld])
    rows_t = table_cat[src].T
    rowmask = (f_ids[:, None] == fld[None, :]).astype(table_cat.dtype)
    t_big_t = (rowmask[:, None, :] * rows_t[None, :, :]).reshape(out_w, v_total)
    t_big_t = t_big_t.astype(jnp.bfloat16)

    x_t = x.T
    bt = _pick_tile(batch)
    out_t = pl.pallas_call(
        functools.partial(_multihot_lookup_t_kernel, vpf=vpf,
                          num_fields=num_fields),
        out_shape=jax.ShapeDtypeStruct((out_w, batch), table_cat.dtype),
        grid=(batch // bt,),
        in_specs=[
            pl.BlockSpec((num_fields, bt), lambda b: (0, b)),
            pl.BlockSpec((out_w, v_total), lambda b: (0, 0)),
        ],
        out_specs=pl.BlockSpec((out_w, bt), lambda b: (0, b)),
        compiler_params=pltpu.CompilerParams(
            dimension_semantics=("parallel",)),
    )(x_t, t_big_t)
    return out_t.reshape(num_fields, emb_dim, batch).transpose(2, 0, 1)
